# baseline (device time: 104844 ns/iter reference)
import jax
import jax.numpy as jnp
from jax import lax
from jax.experimental import pallas as pl
from jax.experimental.pallas import tpu as pltpu

N_HEADS = 16


def _body(q_hbm, k_hbm, v_hbm, o_hbm,
          qs, qb, kb, vb, ks, vs, kr, vr, oa, la, ob,
          load_sems, store_sems, ksend, vsend, krecv, vrecv):
    phase = pl.program_id(0)
    h = pl.program_id(1)
    slot = lax.rem(h, 2)

    my_x = lax.axis_index("x")
    my_y = lax.axis_index("y")
    my_z = lax.axis_index("z")
    other = (1 - my_x, my_y, my_z)

    d = q_hbm.shape[-1]
    scale = d ** -0.5

    def load(t, band, slot_):
        src = (q_hbm, k_hbm, v_hbm)[t]
        dst = (qb, kb, vb)[t]
        return pltpu.make_async_copy(
            src.at[0, :, band, :], dst.at[slot_], load_sems.at[slot_, t])

    def band_rdma(sref, rref, s_sem, r_sem):
        return pltpu.make_async_remote_copy(
            src_ref=sref.at[h], dst_ref=rref.at[h],
            send_sem=s_sem.at[h], recv_sem=r_sem.at[h],
            device_id=other, device_id_type=pl.DeviceIdType.MESH)

    def store(band, slot_):
        return pltpu.make_async_copy(
            ob.at[slot_], o_hbm.at[0, :, band, :], store_sems.at[slot_])

    @pl.when(phase == 0)
    def _phase0():
        @pl.when(h == 0)
        def _():
            for t in range(3):
                load(t, 0, 0).start()
                load(t, 1, 1).start()
            barrier_sem = pltpu.get_barrier_semaphore()
            pl.semaphore_signal(
                barrier_sem, inc=1, device_id=other,
                device_id_type=pl.DeviceIdType.MESH)
            pl.semaphore_wait(barrier_sem, 1)

        for t in range(3):
            load(t, h, slot).wait()

        ks[h] = kb[slot].astype(jnp.bfloat16)
        band_rdma(ks, kr, ksend, krecv).start()
        vs[h] = vb[slot].astype(jnp.bfloat16)
        band_rdma(vs, vr, vsend, vrecv).start()
        qs[h] = (qb[slot] * scale).astype(jnp.bfloat16)

        s0 = lax.dot_general(
            qs[h], ks[h], (((1,), (1,)), ((), ())),
            preferred_element_type=jnp.float32)
        p0 = jnp.exp(s0)
        la[h] = jnp.sum(p0, axis=1, keepdims=True)
        oa[h] = lax.dot_general(
            p0.astype(jnp.bfloat16), vs[h], (((1,), (0,)), ((), ())),
            preferred_element_type=jnp.float32)

        @pl.when(h + 2 < N_HEADS)
        def _():
            for t in range(3):
                load(t, h + 2, slot).start()

    @pl.when(phase == 1)
    def _phase1():
        band_rdma(ks, kr, ksend, krecv).wait()
        band_rdma(vs, vr, vsend, vrecv).wait()

        s1 = lax.dot_general(
            qs[h], kr[h], (((1,), (1,)), ((), ())),
            preferred_element_type=jnp.float32)
        p1 = jnp.exp(s1)
        l1 = jnp.sum(p1, axis=1, keepdims=True)
        o1 = lax.dot_general(
            p1.astype(jnp.bfloat16), vr[h], (((1,), (0,)), ((), ())),
            preferred_element_type=jnp.float32)

        @pl.when(h >= 2)
        def _():
            store(h - 2, slot).wait()
        ob[slot] = (oa[h] + o1) / (la[h] + l1)
        store(h, slot).start()

        @pl.when(h == N_HEADS - 1)
        def _():
            store(h - 1, 1 - slot).wait()
            store(h, slot).wait()


def kernel(Q, K, V):
    _, s_half, h, d = Q.shape

    out = pl.pallas_call(
        _body,
        grid=(2, h),
        out_shape=jax.ShapeDtypeStruct((1, s_half, h, d), jnp.float32),
        in_specs=[pl.BlockSpec(memory_space=pltpu.MemorySpace.HBM)] * 3,
        out_specs=pl.BlockSpec(memory_space=pltpu.MemorySpace.HBM),
        scratch_shapes=[
            pltpu.VMEM((h, s_half, d), jnp.bfloat16),
            pltpu.VMEM((2, s_half, d), jnp.float32),
            pltpu.VMEM((2, s_half, d), jnp.float32),
            pltpu.VMEM((2, s_half, d), jnp.float32),
            pltpu.VMEM((h, s_half, d), jnp.bfloat16),
            pltpu.VMEM((h, s_half, d), jnp.bfloat16),
            pltpu.VMEM((h, s_half, d), jnp.bfloat16),
            pltpu.VMEM((h, s_half, d), jnp.bfloat16),
            pltpu.VMEM((h, s_half, d), jnp.float32),
            pltpu.VMEM((h, s_half, 1), jnp.float32),
            pltpu.VMEM((2, s_half, d), jnp.float32),
            pltpu.SemaphoreType.DMA((2, 3)),
            pltpu.SemaphoreType.DMA((2,)),
            pltpu.SemaphoreType.DMA((h,)),
            pltpu.SemaphoreType.DMA((h,)),
            pltpu.SemaphoreType.DMA((h,)),
            pltpu.SemaphoreType.DMA((h,)),
        ],
        compiler_params=pltpu.CompilerParams(
            collective_id=0, vmem_limit_bytes=62 * 1024 * 1024),
    )(Q, K, V)

    return out


# device time: 104166 ns/iter; 1.0065x vs baseline; 1.0065x over previous
import jax
import jax.numpy as jnp
from jax import lax
from jax.experimental import pallas as pl
from jax.experimental.pallas import tpu as pltpu

N_HEADS = 16


def _body(q_hbm, k_hbm, v_hbm, o_hbm,
          qs, qb, kb, vb, ks, vs, kr, vr, oa, la, ob,
          load_sems, store_sems, ksend, vsend, krecv, vrecv):
    phase = pl.program_id(0)
    h = pl.program_id(1)
    slot = lax.rem(h, 2)

    my_x = lax.axis_index("x")
    my_y = lax.axis_index("y")
    my_z = lax.axis_index("z")
    other = (1 - my_x, my_y, my_z)

    d = q_hbm.shape[-1]
    scale = d ** -0.5

    def load(t, band, slot_):
        src = (k_hbm, v_hbm, q_hbm)[t]
        dst = (kb, vb, qb)[t]
        return pltpu.make_async_copy(
            src.at[0, :, band, :], dst.at[slot_], load_sems.at[slot_, t])

    def band_rdma(sref, rref, s_sem, r_sem):
        return pltpu.make_async_remote_copy(
            src_ref=sref.at[h], dst_ref=rref.at[h],
            send_sem=s_sem.at[h], recv_sem=r_sem.at[h],
            device_id=other, device_id_type=pl.DeviceIdType.MESH)

    def store(band, slot_):
        return pltpu.make_async_copy(
            ob.at[slot_], o_hbm.at[0, :, band, :], store_sems.at[slot_])

    @pl.when(phase == 0)
    def _phase0():
        @pl.when(h == 0)
        def _():
            for t in range(3):
                load(t, 0, 0).start()
                load(t, 1, 1).start()
            barrier_sem = pltpu.get_barrier_semaphore()
            pl.semaphore_signal(
                barrier_sem, inc=1, device_id=other,
                device_id_type=pl.DeviceIdType.MESH)
            pl.semaphore_wait(barrier_sem, 1)

        load(0, h, slot).wait()
        ks[h] = kb[slot].astype(jnp.bfloat16)
        band_rdma(ks, kr, ksend, krecv).start()
        load(1, h, slot).wait()
        vs[h] = vb[slot].astype(jnp.bfloat16)
        band_rdma(vs, vr, vsend, vrecv).start()
        load(2, h, slot).wait()
        qs[h] = (qb[slot] * scale).astype(jnp.bfloat16)

        s0 = lax.dot_general(
            qs[h], ks[h], (((1,), (1,)), ((), ())),
            preferred_element_type=jnp.float32)
        p0 = jnp.exp(s0)
        la[h] = jnp.sum(p0, axis=1, keepdims=True)
        oa[h] = lax.dot_general(
            p0.astype(jnp.bfloat16), vs[h], (((1,), (0,)), ((), ())),
            preferred_element_type=jnp.float32)

        @pl.when(h + 2 < N_HEADS)
        def _():
            for t in range(3):
                load(t, h + 2, slot).start()

    @pl.when(phase == 1)
    def _phase1():
        band_rdma(ks, kr, ksend, krecv).wait()
        band_rdma(vs, vr, vsend, vrecv).wait()

        s1 = lax.dot_general(
            qs[h], kr[h], (((1,), (1,)), ((), ())),
            preferred_element_type=jnp.float32)
        p1 = jnp.exp(s1)
        l1 = jnp.sum(p1, axis=1, keepdims=True)
        o1 = lax.dot_general(
            p1.astype(jnp.bfloat16), vr[h], (((1,), (0,)), ((), ())),
            preferred_element_type=jnp.float32)

        @pl.when(h >= 2)
        def _():
            store(h - 2, slot).wait()
        ob[slot] = (oa[h] + o1) / (la[h] + l1)
        store(h, slot).start()

        @pl.when(h == N_HEADS - 1)
        def _():
            store(h - 1, 1 - slot).wait()
            store(h, slot).wait()


def kernel(Q, K, V):
    _, s_half, h, d = Q.shape

    out = pl.pallas_call(
        _body,
        grid=(2, h),
        out_shape=jax.ShapeDtypeStruct((1, s_half, h, d), jnp.float32),
        in_specs=[pl.BlockSpec(memory_space=pltpu.MemorySpace.HBM)] * 3,
        out_specs=pl.BlockSpec(memory_space=pltpu.MemorySpace.HBM),
        scratch_shapes=[
            pltpu.VMEM((h, s_half, d), jnp.bfloat16),
            pltpu.VMEM((2, s_half, d), jnp.float32),
            pltpu.VMEM((2, s_half, d), jnp.float32),
            pltpu.VMEM((2, s_half, d), jnp.float32),
            pltpu.VMEM((h, s_half, d), jnp.bfloat16),
            pltpu.VMEM((h, s_half, d), jnp.bfloat16),
            pltpu.VMEM((h, s_half, d), jnp.bfloat16),
            pltpu.VMEM((h, s_half, d), jnp.bfloat16),
            pltpu.VMEM((h, s_half, d), jnp.float32),
            pltpu.VMEM((h, s_half, 1), jnp.float32),
            pltpu.VMEM((2, s_half, d), jnp.float32),
            pltpu.SemaphoreType.DMA((2, 3)),
            pltpu.SemaphoreType.DMA((2,)),
            pltpu.SemaphoreType.DMA((h,)),
            pltpu.SemaphoreType.DMA((h,)),
            pltpu.SemaphoreType.DMA((h,)),
            pltpu.SemaphoreType.DMA((h,)),
        ],
        compiler_params=pltpu.CompilerParams(
            collective_id=0, vmem_limit_bytes=62 * 1024 * 1024),
    )(Q, K, V)

    return out
